# Initial kernel scaffold; baseline (speedup 1.0000x reference)
import functools

import jax
import jax.numpy as jnp
from jax import lax
from jax.experimental import pallas as pl
from jax.experimental.pallas import tpu as pltpu
from jax.experimental.pallas import tpu_sc as plsc

N = 50000
E = 800000
NP = 50176
RPT = NP // 16
RB = RPT // 4
CHUNK = 256
NCH = E // CHUNK
NW = 32
BT = 2000
GRID = N // BT

_f32 = jnp.float32
_i32 = jnp.int32

_mesh = plsc.VectorSubcoreMesh(core_axis_name="c", subcore_axis_name="s")


def _leaky(a):
    return jnp.maximum(a, 0.0) + 0.2 * jnp.minimum(a, 0.0)


def _l1_body(x_hbm, src_hbm, dst_hbm, w_hbm, z2_hbm, out_hbm,
             x_v, w_v, src_v, dst_v, rows_v, zbuf_v, acc_sh):
    cid = lax.axis_index("c")
    sid = lax.axis_index("s")
    wid = sid * 2 + cid

    pltpu.sync_copy(x_hbm, x_v)
    pltpu.sync_copy(w_hbm, w_v)
    pltpu.sync_copy(z2_hbm, zbuf_v)
    pltpu.sync_copy(z2_hbm.at[pl.ds(0, CHUNK)], rows_v)

    for p in range(4):
        pltpu.sync_copy(zbuf_v, acc_sh.at[pl.ds(sid * RPT + p * RB, RB)])
    plsc.subcore_barrier()

    nt = (NCH - wid + NW - 1) // NW

    def chunk_body(t, carry):
        start = (wid + t * NW) * CHUNK
        pltpu.sync_copy(src_hbm.at[pl.ds(start, CHUNK)], src_v)
        pltpu.sync_copy(dst_hbm.at[pl.ds(start, CHUNK)], dst_v)
        for q in range(CHUNK // 16):
            si = src_v[pl.ds(q * 16, 16)]
            di = dst_v[pl.ds(q * 16, 16)]
            xs = plsc.load_gather(x_v, [si])
            xd = plsc.load_gather(x_v, [di])
            rid = lax.broadcasted_iota(_i32, (16,), 0) + q * 16
            for hd in range(4):
                a = xs * w_v[hd] + xd * w_v[4 + hd]
                e = jnp.exp(_leaky(a))
                plsc.store_scatter(rows_v, [rid, jnp.full((16,), hd, _i32)], e)
                plsc.store_scatter(
                    rows_v, [rid, jnp.full((16,), hd + 4, _i32)], e * xs)
        pltpu.sync_copy(rows_v, acc_sh.at[dst_v], add=True)
        return carry

    lax.fori_loop(0, nt, chunk_body, 0)
    plsc.subcore_barrier()

    for p in range(4):
        r0 = sid * RPT + p * RB
        pltpu.sync_copy(acc_sh.at[pl.ds(r0, RB)], zbuf_v)
        pltpu.sync_copy(zbuf_v, out_hbm.at[cid, pl.ds(r0, RB)])


_l1 = pl.kernel(
    _l1_body,
    out_type=jax.ShapeDtypeStruct((2, NP, 16), _f32),
    mesh=_mesh,
    scratch_types=[
        pltpu.VMEM((NP,), _f32),
        pltpu.VMEM((16,), _f32),
        pltpu.VMEM((CHUNK,), _i32),
        pltpu.VMEM((CHUNK,), _i32),
        pltpu.VMEM((CHUNK, 16), _f32),
        pltpu.VMEM((RB, 16), _f32),
        pltpu.VMEM_SHARED((NP, 16), _f32),
    ],
)


def _mid_body(p_ref, x_ref, sd_ref, w1_ref, b1_ref, w2_ref, a2_ref,
              g_ref, aa_ref, mx_ref):
    i = pl.program_id(0)
    xb = x_ref[...]
    den = p_ref[0, :, 0:4] + p_ref[1, :, 0:4]
    num = p_ref[0, :, 4:8] + p_ref[1, :, 4:8]
    sd = sd_ref[...]
    a_self = xb * (sd[0:1, :] + sd[1:2, :])
    e_self = jnp.exp(_leaky(a_self))
    den = den + e_self + 1e-16
    num = num + e_self * xb
    s = num / den
    g = jnp.zeros((BT, 16), _f32)
    for hd in range(4):
        t = s[:, hd:hd + 1] * w1_ref[hd:hd + 1, :] + b1_ref[hd:hd + 1, :]
        t = jnp.where(t > 0, t, jnp.expm1(t))
        g = g + jnp.dot(t, w2_ref[hd * 32:(hd + 1) * 32, :],
                        preferred_element_type=_f32)
    g_ref[...] = g
    aa = jnp.dot(g, a2_ref[...], preferred_element_type=_f32)
    aa_ref[...] = aa
    m = jnp.max(aa, axis=0, keepdims=True)

    @pl.when(i == 0)
    def _():
        mx_ref[...] = m

    @pl.when(i > 0)
    def _():
        mx_ref[...] = jnp.maximum(mx_ref[...], m)


_mid = pl.pallas_call(
    _mid_body,
    grid=(GRID,),
    in_specs=[
        pl.BlockSpec((2, BT, 16), lambda i: (0, i, 0)),
        pl.BlockSpec((BT, 1), lambda i: (i, 0)),
        pl.BlockSpec((2, 4), lambda i: (0, 0)),
        pl.BlockSpec((4, 32), lambda i: (0, 0)),
        pl.BlockSpec((4, 32), lambda i: (0, 0)),
        pl.BlockSpec((128, 16), lambda i: (0, 0)),
        pl.BlockSpec((16, 2), lambda i: (0, 0)),
    ],
    out_specs=[
        pl.BlockSpec((BT, 16), lambda i: (i, 0)),
        pl.BlockSpec((BT, 2), lambda i: (i, 0)),
        pl.BlockSpec((1, 2), lambda i: (0, 0)),
    ],
    out_shape=[
        jax.ShapeDtypeStruct((N, 16), _f32),
        jax.ShapeDtypeStruct((N, 2), _f32),
        jax.ShapeDtypeStruct((1, 2), _f32),
    ],
)


def _l2_body(g_hbm, as_hbm, ad_hbm, src_hbm, dst_hbm, m_hbm, z2_hbm, z1_hbm,
             outr_hbm, outd_hbm,
             as_v, ad_v, m_v, src_v, dst_v, rows_v, ex_v, zbuf_v, zbufd_v,
             acc_sh, den_sh, sem):
    cid = lax.axis_index("c")
    sid = lax.axis_index("s")
    wid = sid * 2 + cid

    pltpu.sync_copy(as_hbm, as_v)
    pltpu.sync_copy(ad_hbm, ad_v)
    pltpu.sync_copy(m_hbm, m_v)
    pltpu.sync_copy(z2_hbm, zbuf_v)
    pltpu.sync_copy(z1_hbm.at[pl.ds(0, RB)], zbufd_v)
    m2 = _leaky(m_v[0] + m_v[1])

    for p in range(4):
        r0 = sid * RPT + p * RB
        pltpu.sync_copy(zbuf_v, acc_sh.at[pl.ds(r0, RB)])
        pltpu.sync_copy(zbufd_v, den_sh.at[pl.ds(r0, RB)])
    plsc.subcore_barrier()

    nt = (NCH - wid + NW - 1) // NW

    def chunk_body(t, carry):
        start = (wid + t * NW) * CHUNK
        pltpu.sync_copy(src_hbm.at[pl.ds(start, CHUNK)], src_v)
        pltpu.sync_copy(dst_hbm.at[pl.ds(start, CHUNK)], dst_v)
        pltpu.async_copy(g_hbm.at[src_v], rows_v, sem).wait()
        for q in range(CHUNK // 16):
            si = src_v[pl.ds(q * 16, 16)]
            di = dst_v[pl.ds(q * 16, 16)]
            a = plsc.load_gather(as_v, [si]) + plsc.load_gather(ad_v, [di])
            ex_v[pl.ds(q * 16, 16)] = jnp.exp(_leaky(a) - m2)
        for j in range(CHUNK):
            eb = plsc.load_gather(ex_v, [jnp.full((16,), j, _i32)])
            rows_v[j, :] = rows_v[j, :] * eb
        pltpu.sync_copy(rows_v, acc_sh.at[dst_v], add=True)
        pltpu.sync_copy(ex_v, den_sh.at[dst_v], add=True)
        return carry

    lax.fori_loop(0, nt, chunk_body, 0)
    plsc.subcore_barrier()

    for p in range(4):
        r0 = sid * RPT + p * RB
        pltpu.sync_copy(acc_sh.at[pl.ds(r0, RB)], zbuf_v)
        pltpu.sync_copy(zbuf_v, outr_hbm.at[cid, pl.ds(r0, RB)])
        pltpu.sync_copy(den_sh.at[pl.ds(r0, RB)], zbufd_v)
        pltpu.sync_copy(zbufd_v, outd_hbm.at[cid, pl.ds(r0, RB)])


_l2 = pl.kernel(
    _l2_body,
    out_type=[
        jax.ShapeDtypeStruct((2, NP, 16), _f32),
        jax.ShapeDtypeStruct((2, NP), _f32),
    ],
    mesh=_mesh,
    scratch_types=[
        pltpu.VMEM((N,), _f32),
        pltpu.VMEM((N,), _f32),
        pltpu.VMEM((8,), _f32),
        pltpu.VMEM((CHUNK,), _i32),
        pltpu.VMEM((CHUNK,), _i32),
        pltpu.VMEM((CHUNK, 16), _f32),
        pltpu.VMEM((CHUNK,), _f32),
        pltpu.VMEM((RB, 16), _f32),
        pltpu.VMEM((RB,), _f32),
        pltpu.VMEM_SHARED((NP, 16), _f32),
        pltpu.VMEM_SHARED((NP,), _f32),
        pltpu.SemaphoreType.DMA,
    ],
)


def _fin_body(pr_ref, pd_ref, g_ref, aa_ref, mx_ref, b2_ref, o_ref):
    m2 = _leaky(mx_ref[0, 0] + mx_ref[0, 1])
    aa = aa_ref[...]
    a2s = _leaky(aa[:, 0:1] + aa[:, 1:2])
    es = jnp.exp(a2s - m2)
    g = g_ref[...]
    num = pr_ref[0] + pr_ref[1] + es * g
    den = pd_ref[0] + pd_ref[1] + es + 1e-16
    o_ref[...] = num / den + b2_ref[...]


_fin = pl.pallas_call(
    _fin_body,
    grid=(GRID,),
    in_specs=[
        pl.BlockSpec((2, BT, 16), lambda i: (0, i, 0)),
        pl.BlockSpec((2, BT, 1), lambda i: (0, i, 0)),
        pl.BlockSpec((BT, 16), lambda i: (i, 0)),
        pl.BlockSpec((BT, 2), lambda i: (i, 0)),
        pl.BlockSpec((1, 2), lambda i: (0, 0)),
        pl.BlockSpec((1, 16), lambda i: (0, 0)),
    ],
    out_specs=pl.BlockSpec((BT, 16), lambda i: (i, 0)),
    out_shape=jax.ShapeDtypeStruct((N, 16), _f32),
)


def kernel(x, edge_index, W1, att_src1, att_dst1, b1, W2, att_src2, att_dst2,
           b2):
    src = edge_index[0].astype(_i32)
    dst = edge_index[1].astype(_i32)
    xf = x[:, 0]
    x_pad = jnp.pad(xf, (0, NP - N))
    W1r = W1.reshape(4, 32)
    s1 = (W1r * att_src1[0]).sum(-1)
    d1 = (W1r * att_dst1[0]).sum(-1)
    wvec = jnp.concatenate([s1, d1, jnp.zeros((8,), _f32)])
    z2 = jnp.zeros((RB, 16), _f32)
    z1 = jnp.zeros((RB,), _f32)

    part1 = _l1(x_pad, src, dst, wvec, z2)

    sd = jnp.stack([s1, d1])
    att2 = jnp.stack([att_src2[0, 0], att_dst2[0, 0]], axis=1)
    g, aa, mx = _mid(part1, x, sd, W1r, b1.reshape(4, 32), W2.T, att2)

    as2 = jnp.ascontiguousarray(aa[:, 0])
    ad2 = jnp.ascontiguousarray(aa[:, 1])
    m8 = jnp.pad(mx.reshape(2), (0, 6))
    part2r, part2d = _l2(g, as2, ad2, src, dst, m8, z2, z1)

    return _fin(part2r, part2d.reshape(2, NP, 1), g, aa, mx,
                b2.reshape(1, 16))


# speedup vs baseline: 128.4666x; 128.4666x over previous
import functools

import jax
import jax.numpy as jnp
from jax import lax
from jax.experimental import pallas as pl
from jax.experimental.pallas import tpu as pltpu
from jax.experimental.pallas import tpu_sc as plsc

N = 50000
E = 800000
NP = 50176
RPT = NP // 16
RB = RPT // 4
CHUNK = 256
NCH = E // CHUNK
NW = 32
BT = 2000
GRID = N // BT

_f32 = jnp.float32
_i32 = jnp.int32

_mesh = plsc.VectorSubcoreMesh(core_axis_name="c", subcore_axis_name="s")
_sc_params = pltpu.CompilerParams(
    needs_layout_passes=False, use_tc_tiling_on_sc=False)


def _leaky(a):
    return jnp.maximum(a, 0.0) + 0.2 * jnp.minimum(a, 0.0)


def _l1_body(x_hbm, src_hbm, dst_hbm, w_hbm, z2_hbm, out_hbm,
             x_v, w_v, src_v, dst_v, rows_v, zbuf_v, acc_sh):
    cid = lax.axis_index("c")
    sid = lax.axis_index("s")
    wid = sid * 2 + cid

    pltpu.sync_copy(x_hbm, x_v)
    pltpu.sync_copy(w_hbm, w_v)
    pltpu.sync_copy(z2_hbm, zbuf_v)
    pltpu.sync_copy(z2_hbm.at[pl.ds(0, CHUNK)], rows_v)

    for p in range(4):
        pltpu.sync_copy(zbuf_v, acc_sh.at[pl.ds(sid * RPT + p * RB, RB)])
    plsc.subcore_barrier()

    wv = w_v[...]
    nt = (NCH - wid + NW - 1) // NW

    def chunk_body(t, carry):
        start = (wid + t * NW) * CHUNK
        pltpu.sync_copy(src_hbm.at[pl.ds(start, CHUNK)], src_v)
        pltpu.sync_copy(dst_hbm.at[pl.ds(start, CHUNK)], dst_v)
        for q in range(CHUNK // 16):
            si = src_v[pl.ds(q * 16, 16)]
            di = dst_v[pl.ds(q * 16, 16)]
            xs = plsc.load_gather(x_v, [si])
            xd = plsc.load_gather(x_v, [di])
            rid = lax.broadcasted_iota(_i32, (16,), 0) + q * 16
            for hd in range(4):
                a = xs * wv[hd] + xd * wv[4 + hd]
                e = jnp.exp(_leaky(a))
                plsc.store_scatter(rows_v, [rid, jnp.full((16,), hd, _i32)], e)
                plsc.store_scatter(
                    rows_v, [rid, jnp.full((16,), hd + 4, _i32)], e * xs)
        pltpu.sync_copy(rows_v, acc_sh.at[dst_v], add=True)
        return carry

    lax.fori_loop(0, nt, chunk_body, 0)
    plsc.subcore_barrier()

    for p in range(4):
        r0 = sid * RPT + p * RB
        pltpu.sync_copy(acc_sh.at[pl.ds(r0, RB)], zbuf_v)
        pltpu.sync_copy(zbuf_v, out_hbm.at[cid, pl.ds(r0, RB)])


_l1 = pl.kernel(
    _l1_body,
    out_type=jax.ShapeDtypeStruct((2, NP, 16), _f32),
    mesh=_mesh,
    scratch_types=[
        pltpu.VMEM((NP,), _f32),
        pltpu.VMEM((16,), _f32),
        pltpu.VMEM((CHUNK,), _i32),
        pltpu.VMEM((CHUNK,), _i32),
        pltpu.VMEM((CHUNK, 16), _f32),
        pltpu.VMEM((RB, 16), _f32),
        pltpu.VMEM_SHARED((NP, 16), _f32),
    ],
    compiler_params=_sc_params,
)


def _mid_body(p_ref, x_ref, sd_ref, w1_ref, b1_ref, w2_ref, a2_ref,
              g_ref, aa_ref, mx_ref):
    i = pl.program_id(0)
    xb = x_ref[...]
    den = p_ref[0, :, 0:4] + p_ref[1, :, 0:4]
    num = p_ref[0, :, 4:8] + p_ref[1, :, 4:8]
    sd = sd_ref[...]
    a_self = xb * (sd[0:1, :] + sd[1:2, :])
    e_self = jnp.exp(_leaky(a_self))
    den = den + e_self + 1e-16
    num = num + e_self * xb
    s = num / den
    g = jnp.zeros((BT, 16), _f32)
    for hd in range(4):
        t = s[:, hd:hd + 1] * w1_ref[hd:hd + 1, :] + b1_ref[hd:hd + 1, :]
        t = jnp.where(t > 0, t, jnp.exp(jnp.minimum(t, 0.0)) - 1.0)
        g = g + jnp.dot(t, w2_ref[hd * 32:(hd + 1) * 32, :],
                        preferred_element_type=_f32)
    g_ref[...] = g
    aa = jnp.dot(g, a2_ref[...], preferred_element_type=_f32)
    aa_ref[...] = aa
    m = jnp.max(aa, axis=0, keepdims=True)

    @pl.when(i == 0)
    def _():
        mx_ref[...] = m

    @pl.when(i > 0)
    def _():
        mx_ref[...] = jnp.maximum(mx_ref[...], m)


_mid = pl.pallas_call(
    _mid_body,
    grid=(GRID,),
    in_specs=[
        pl.BlockSpec((2, BT, 16), lambda i: (0, i, 0)),
        pl.BlockSpec((BT, 1), lambda i: (i, 0)),
        pl.BlockSpec((2, 4), lambda i: (0, 0)),
        pl.BlockSpec((4, 32), lambda i: (0, 0)),
        pl.BlockSpec((4, 32), lambda i: (0, 0)),
        pl.BlockSpec((128, 16), lambda i: (0, 0)),
        pl.BlockSpec((16, 2), lambda i: (0, 0)),
    ],
    out_specs=[
        pl.BlockSpec((BT, 16), lambda i: (i, 0)),
        pl.BlockSpec((BT, 2), lambda i: (i, 0)),
        pl.BlockSpec((1, 2), lambda i: (0, 0)),
    ],
    out_shape=[
        jax.ShapeDtypeStruct((N, 16), _f32),
        jax.ShapeDtypeStruct((N, 2), _f32),
        jax.ShapeDtypeStruct((1, 2), _f32),
    ],
)


def _l2_body(g_hbm, as_hbm, ad_hbm, src_hbm, dst_hbm, m_hbm, z2_hbm, z1_hbm,
             outr_hbm, outd_hbm,
             avs_v, avd_v, m_v, src_v, dst_v, rows_v, ex_v, zbuf_v, zbufd_v,
             acc_sh, den_sh, sem):
    cid = lax.axis_index("c")
    sid = lax.axis_index("s")
    wid = sid * 2 + cid

    pltpu.sync_copy(m_hbm, m_v)
    pltpu.sync_copy(z2_hbm, zbuf_v)
    pltpu.sync_copy(z1_hbm.at[pl.ds(0, RB)], zbufd_v)
    mv = m_v[...]
    m2 = _leaky(mv[0] + mv[1])

    for p in range(4):
        r0 = sid * RPT + p * RB
        pltpu.sync_copy(zbuf_v, acc_sh.at[pl.ds(r0, RB)])
        pltpu.sync_copy(zbufd_v, den_sh.at[pl.ds(r0, RB)])
    plsc.subcore_barrier()

    nt = (NCH - wid + NW - 1) // NW

    def chunk_body(t, carry):
        start = (wid + t * NW) * CHUNK
        pltpu.sync_copy(src_hbm.at[pl.ds(start, CHUNK)], src_v)
        pltpu.sync_copy(dst_hbm.at[pl.ds(start, CHUNK)], dst_v)
        pltpu.async_copy(g_hbm.at[src_v], rows_v, sem).wait()
        pltpu.async_copy(as_hbm.at[src_v], avs_v, sem).wait()
        pltpu.async_copy(ad_hbm.at[dst_v], avd_v, sem).wait()
        for q in range(CHUNK // 16):
            a = avs_v[pl.ds(q * 16, 16)] + avd_v[pl.ds(q * 16, 16)]
            ex_v[pl.ds(q * 16, 16)] = jnp.exp(_leaky(a) - m2)
        for j in range(CHUNK):
            eb = plsc.load_gather(ex_v, [jnp.full((16,), j, _i32)])
            rows_v[j, :] = rows_v[j, :] * eb
        pltpu.sync_copy(rows_v, acc_sh.at[dst_v], add=True)
        pltpu.sync_copy(ex_v, den_sh.at[dst_v], add=True)
        return carry

    lax.fori_loop(0, nt, chunk_body, 0)
    plsc.subcore_barrier()

    for p in range(4):
        r0 = sid * RPT + p * RB
        pltpu.sync_copy(acc_sh.at[pl.ds(r0, RB)], zbuf_v)
        pltpu.sync_copy(zbuf_v, outr_hbm.at[cid, pl.ds(r0, RB)])
        pltpu.sync_copy(den_sh.at[pl.ds(r0, RB)], zbufd_v)
        pltpu.sync_copy(zbufd_v, outd_hbm.at[cid, pl.ds(r0, RB)])


_l2 = pl.kernel(
    _l2_body,
    out_type=[
        jax.ShapeDtypeStruct((2, NP, 16), _f32),
        jax.ShapeDtypeStruct((2, NP), _f32),
    ],
    mesh=_mesh,
    scratch_types=[
        pltpu.VMEM((CHUNK,), _f32),
        pltpu.VMEM((CHUNK,), _f32),
        pltpu.VMEM((16,), _f32),
        pltpu.VMEM((CHUNK,), _i32),
        pltpu.VMEM((CHUNK,), _i32),
        pltpu.VMEM((CHUNK, 16), _f32),
        pltpu.VMEM((CHUNK,), _f32),
        pltpu.VMEM((RB, 16), _f32),
        pltpu.VMEM((RB,), _f32),
        pltpu.VMEM_SHARED((NP, 16), _f32),
        pltpu.VMEM_SHARED((NP,), _f32),
        pltpu.SemaphoreType.DMA,
    ],
    compiler_params=_sc_params,
)


def _fin_body(pr_ref, pd_ref, g_ref, aa_ref, mx_ref, b2_ref, o_ref):
    m2 = _leaky(mx_ref[0, 0] + mx_ref[0, 1])
    aa = aa_ref[...]
    a2s = _leaky(aa[:, 0:1] + aa[:, 1:2])
    es = jnp.exp(a2s - m2)
    g = g_ref[...]
    num = pr_ref[0] + pr_ref[1] + es * g
    den = pd_ref[0] + pd_ref[1] + es + 1e-16
    o_ref[...] = num / den + b2_ref[...]


_fin = pl.pallas_call(
    _fin_body,
    grid=(GRID,),
    in_specs=[
        pl.BlockSpec((2, BT, 16), lambda i: (0, i, 0)),
        pl.BlockSpec((2, BT, 1), lambda i: (0, i, 0)),
        pl.BlockSpec((BT, 16), lambda i: (i, 0)),
        pl.BlockSpec((BT, 2), lambda i: (i, 0)),
        pl.BlockSpec((1, 2), lambda i: (0, 0)),
        pl.BlockSpec((1, 16), lambda i: (0, 0)),
    ],
    out_specs=pl.BlockSpec((BT, 16), lambda i: (i, 0)),
    out_shape=jax.ShapeDtypeStruct((N, 16), _f32),
)


def kernel(x, edge_index, W1, att_src1, att_dst1, b1, W2, att_src2, att_dst2,
           b2):
    src = edge_index[0].astype(_i32)
    dst = edge_index[1].astype(_i32)
    xf = x[:, 0]
    x_pad = jnp.pad(xf, (0, NP - N))
    W1r = W1.reshape(4, 32)
    s1 = (W1r * att_src1[0]).sum(-1)
    d1 = (W1r * att_dst1[0]).sum(-1)
    wvec = jnp.concatenate([s1, d1, jnp.zeros((8,), _f32)])
    z2 = jnp.zeros((RB, 16), _f32)
    z1 = jnp.zeros((RB,), _f32)

    part1 = _l1(x_pad, src, dst, wvec, z2)

    sd = jnp.stack([s1, d1])
    att2 = jnp.stack([att_src2[0, 0], att_dst2[0, 0]], axis=1)
    g, aa, mx = _mid(part1, x, sd, W1r, b1.reshape(4, 32), W2.T, att2)

    as2 = aa[:, 0] + 0.0
    ad2 = aa[:, 1] + 0.0
    m8 = jnp.pad(mx.reshape(2), (0, 14))
    part2r, part2d = _l2(g, as2, ad2, src, dst, m8, z2, z1)

    return _fin(part2r, part2d.reshape(2, NP, 1), g, aa, mx,
                b2.reshape(1, 16))


# speedup vs baseline: 186.4525x; 1.4514x over previous
import jax
import jax.numpy as jnp
from jax import lax
from jax.experimental import pallas as pl
from jax.experimental.pallas import tpu as pltpu
from jax.experimental.pallas import tpu_sc as plsc

N = 50000
E = 800000
NP = 50176
RPT = NP // 16
RB = RPT // 4
CHUNK = 256
NCH = E // CHUNK
NW = 32
BT = 2000
GRID = N // BT

_f32 = jnp.float32
_i32 = jnp.int32

_mesh = plsc.VectorSubcoreMesh(core_axis_name="c", subcore_axis_name="s")
_sc_params = pltpu.CompilerParams(
    needs_layout_passes=False, use_tc_tiling_on_sc=False)


def _leaky(a):
    return jnp.maximum(a, 0.0) + 0.2 * jnp.minimum(a, 0.0)


def _l1_body(x_hbm, src_hbm, dst_hbm, w_hbm, z8_hbm, out_hbm,
             x_v, w_v, src0_v, dst0_v, src1_v, dst1_v, rows_v, zbuf_v,
             acc_sh, asem0, asem1):
    cid = lax.axis_index("c")
    sid = lax.axis_index("s")
    wid = sid * 2 + cid

    pltpu.sync_copy(x_hbm, x_v)
    pltpu.sync_copy(w_hbm, w_v)
    pltpu.sync_copy(z8_hbm, zbuf_v)

    for p in range(4):
        pltpu.sync_copy(zbuf_v, acc_sh.at[pl.ds(sid * RPT + p * RB, RB)])
    plsc.subcore_barrier()

    wv = w_v[...]
    nt = (NCH - wid + NW - 1) // NW

    def fire_idx(t, sv, dv, sem):
        start = (wid + t * NW) * CHUNK
        pltpu.async_copy(src_hbm.at[pl.ds(start, CHUNK)], sv, sem)
        pltpu.async_copy(dst_hbm.at[pl.ds(start, CHUNK)], dv, sem)

    def process(sv, dv, sem):
        pltpu.make_async_copy(src_hbm.at[pl.ds(0, CHUNK)], sv, sem).wait()
        pltpu.make_async_copy(dst_hbm.at[pl.ds(0, CHUNK)], dv, sem).wait()
        for q in range(CHUNK // 16):
            si = sv[pl.ds(q * 16, 16)]
            di = dv[pl.ds(q * 16, 16)]
            xs = plsc.load_gather(x_v, [si])
            xd = plsc.load_gather(x_v, [di])
            rid = lax.broadcasted_iota(_i32, (16,), 0) + q * 16
            for hd in range(4):
                a = xs * wv[hd] + xd * wv[4 + hd]
                e = jnp.exp(_leaky(a))
                plsc.store_scatter(rows_v, [rid, jnp.full((16,), hd, _i32)], e)
                plsc.store_scatter(
                    rows_v, [rid, jnp.full((16,), hd + 4, _i32)], e * xs)
        pltpu.sync_copy(rows_v, acc_sh.at[dv], add=True)

    fire_idx(0, src0_v, dst0_v, asem0)

    @pl.when(nt > 1)
    def _():
        fire_idx(1, src1_v, dst1_v, asem1)

    def pair_body(u, carry):
        t0 = 2 * u
        t1 = t0 + 1

        @pl.when(t0 < nt)
        def _():
            process(src0_v, dst0_v, asem0)

            @pl.when(t0 + 2 < nt)
            def _():
                fire_idx(t0 + 2, src0_v, dst0_v, asem0)

        @pl.when(t1 < nt)
        def _():
            process(src1_v, dst1_v, asem1)

            @pl.when(t1 + 2 < nt)
            def _():
                fire_idx(t1 + 2, src1_v, dst1_v, asem1)

        return carry

    lax.fori_loop(0, (nt + 1) // 2, pair_body, 0)
    plsc.subcore_barrier()

    for p in range(4):
        r0 = sid * RPT + p * RB
        pltpu.sync_copy(acc_sh.at[pl.ds(r0, RB)], zbuf_v)
        pltpu.sync_copy(zbuf_v, out_hbm.at[cid, pl.ds(r0, RB)])


_l1 = pl.kernel(
    _l1_body,
    out_type=jax.ShapeDtypeStruct((2, NP, 8), _f32),
    mesh=_mesh,
    scratch_types=[
        pltpu.VMEM((N,), _f32),
        pltpu.VMEM((16,), _f32),
        pltpu.VMEM((CHUNK,), _i32),
        pltpu.VMEM((CHUNK,), _i32),
        pltpu.VMEM((CHUNK,), _i32),
        pltpu.VMEM((CHUNK,), _i32),
        pltpu.VMEM((CHUNK, 8), _f32),
        pltpu.VMEM((RB, 8), _f32),
        pltpu.VMEM_SHARED((NP, 8), _f32),
        pltpu.SemaphoreType.DMA,
        pltpu.SemaphoreType.DMA,
    ],
    compiler_params=_sc_params,
)


def _mid_body(p_ref, x_ref, sd_ref, w1_ref, b1_ref, w2_ref, a2_ref,
              g_ref, aa_ref, mx_ref):
    i = pl.program_id(0)
    xb = x_ref[...]
    den = p_ref[0, :, 0:4] + p_ref[1, :, 0:4]
    num = p_ref[0, :, 4:8] + p_ref[1, :, 4:8]
    sd = sd_ref[...]
    a_self = xb * (sd[0:1, :] + sd[1:2, :])
    e_self = jnp.exp(_leaky(a_self))
    den = den + e_self + 1e-16
    num = num + e_self * xb
    s = num / den
    g = jnp.zeros((BT, 16), _f32)
    for hd in range(4):
        t = s[:, hd:hd + 1] * w1_ref[hd:hd + 1, :] + b1_ref[hd:hd + 1, :]
        t = jnp.where(t > 0, t, jnp.exp(jnp.minimum(t, 0.0)) - 1.0)
        g = g + lax.dot_general(
            t, w2_ref[:, hd * 32:(hd + 1) * 32],
            (((1,), (1,)), ((), ())), preferred_element_type=_f32,
            precision=lax.Precision.HIGHEST)
    g_ref[...] = g
    aa = jnp.dot(g, a2_ref[...], preferred_element_type=_f32,
                 precision=lax.Precision.HIGHEST)
    aa_ref[...] = aa
    m = jnp.concatenate(
        [jnp.max(aa, axis=0, keepdims=True), jnp.zeros((1, 14), _f32)],
        axis=1)

    @pl.when(i == 0)
    def _():
        mx_ref[...] = m

    @pl.when(i > 0)
    def _():
        mx_ref[...] = jnp.maximum(mx_ref[...], m)


_mid = pl.pallas_call(
    _mid_body,
    grid=(GRID,),
    in_specs=[
        pl.BlockSpec((2, BT, 8), lambda i: (0, i, 0)),
        pl.BlockSpec((BT, 1), lambda i: (i, 0)),
        pl.BlockSpec((2, 4), lambda i: (0, 0)),
        pl.BlockSpec((4, 32), lambda i: (0, 0)),
        pl.BlockSpec((4, 32), lambda i: (0, 0)),
        pl.BlockSpec((16, 128), lambda i: (0, 0)),
        pl.BlockSpec((16, 2), lambda i: (0, 0)),
    ],
    out_specs=[
        pl.BlockSpec((BT, 16), lambda i: (i, 0)),
        pl.BlockSpec((BT, 2), lambda i: (i, 0)),
        pl.BlockSpec((1, 16), lambda i: (0, 0)),
    ],
    out_shape=[
        jax.ShapeDtypeStruct((N, 16), _f32),
        jax.ShapeDtypeStruct((N, 2), _f32),
        jax.ShapeDtypeStruct((1, 16), _f32),
    ],
)


def _l2_body(g_hbm, as_hbm, ad_hbm, src_hbm, dst_hbm, m_hbm, z16_hbm, z1_hbm,
             outr_hbm, outd_hbm,
             m_v, src0_v, dst0_v, src1_v, dst1_v, rows0_v, rows1_v,
             avs0_v, avd0_v, avs1_v, avd1_v, ex0_v, ex1_v, zbuf_v, zbufd_v,
             acc_sh, den_sh, gsem0, gsem1):
    cid = lax.axis_index("c")
    sid = lax.axis_index("s")
    wid = sid * 2 + cid

    pltpu.sync_copy(m_hbm, m_v)
    pltpu.sync_copy(z16_hbm, zbuf_v)
    pltpu.sync_copy(z1_hbm.at[pl.ds(0, RB)], zbufd_v)
    mv = m_v[...]
    m2 = _leaky(mv[0] + mv[1])

    for p in range(4):
        r0 = sid * RPT + p * RB
        pltpu.sync_copy(zbuf_v, acc_sh.at[pl.ds(r0, RB)])
        pltpu.sync_copy(zbufd_v, den_sh.at[pl.ds(r0, RB)])
    plsc.subcore_barrier()

    nt = (NCH - wid + NW - 1) // NW

    def fire(t, sv, dv, rows, avs, avd, sem):
        start = (wid + t * NW) * CHUNK
        pltpu.sync_copy(src_hbm.at[pl.ds(start, CHUNK)], sv)
        pltpu.sync_copy(dst_hbm.at[pl.ds(start, CHUNK)], dv)
        pltpu.async_copy(g_hbm.at[sv], rows, sem)
        pltpu.async_copy(as_hbm.at[sv], avs, sem)
        pltpu.async_copy(ad_hbm.at[dv], avd, sem)

    def process(sv, dv, rows, avs, avd, ex, sem):
        pltpu.make_async_copy(g_hbm.at[sv], rows, sem).wait()
        pltpu.make_async_copy(as_hbm.at[sv], avs, sem).wait()
        pltpu.make_async_copy(ad_hbm.at[dv], avd, sem).wait()
        for q in range(CHUNK // 16):
            a = avs[pl.ds(q * 16, 16)] + avd[pl.ds(q * 16, 16)]
            e16 = jnp.exp(_leaky(a) - m2)
            ex[pl.ds(q * 16, 16)] = e16
            for j in range(16):
                r = q * 16 + j
                rows[r, :] = rows[r, :] * e16[j]
        pltpu.sync_copy(rows, acc_sh.at[dv], add=True)
        pltpu.sync_copy(ex, den_sh.at[dv], add=True)

    fire(0, src0_v, dst0_v, rows0_v, avs0_v, avd0_v, gsem0)

    @pl.when(nt > 1)
    def _():
        fire(1, src1_v, dst1_v, rows1_v, avs1_v, avd1_v, gsem1)

    def pair_body(u, carry):
        t0 = 2 * u
        t1 = t0 + 1

        @pl.when(t0 < nt)
        def _():
            process(src0_v, dst0_v, rows0_v, avs0_v, avd0_v, ex0_v, gsem0)

            @pl.when(t0 + 2 < nt)
            def _():
                fire(t0 + 2, src0_v, dst0_v, rows0_v, avs0_v, avd0_v, gsem0)

        @pl.when(t1 < nt)
        def _():
            process(src1_v, dst1_v, rows1_v, avs1_v, avd1_v, ex1_v, gsem1)

            @pl.when(t1 + 2 < nt)
            def _():
                fire(t1 + 2, src1_v, dst1_v, rows1_v, avs1_v, avd1_v, gsem1)

        return carry

    lax.fori_loop(0, (nt + 1) // 2, pair_body, 0)
    plsc.subcore_barrier()

    for p in range(4):
        r0 = sid * RPT + p * RB
        pltpu.sync_copy(acc_sh.at[pl.ds(r0, RB)], zbuf_v)
        pltpu.sync_copy(zbuf_v, outr_hbm.at[cid, pl.ds(r0, RB)])
        pltpu.sync_copy(den_sh.at[pl.ds(r0, RB)], zbufd_v)
        pltpu.sync_copy(zbufd_v, outd_hbm.at[cid, pl.ds(r0, RB)])


_l2 = pl.kernel(
    _l2_body,
    out_type=[
        jax.ShapeDtypeStruct((2, NP, 16), _f32),
        jax.ShapeDtypeStruct((2, NP), _f32),
    ],
    mesh=_mesh,
    scratch_types=[
        pltpu.VMEM((16,), _f32),
        pltpu.VMEM((CHUNK,), _i32),
        pltpu.VMEM((CHUNK,), _i32),
        pltpu.VMEM((CHUNK,), _i32),
        pltpu.VMEM((CHUNK,), _i32),
        pltpu.VMEM((CHUNK, 16), _f32),
        pltpu.VMEM((CHUNK, 16), _f32),
        pltpu.VMEM((CHUNK,), _f32),
        pltpu.VMEM((CHUNK,), _f32),
        pltpu.VMEM((CHUNK,), _f32),
        pltpu.VMEM((CHUNK,), _f32),
        pltpu.VMEM((CHUNK,), _f32),
        pltpu.VMEM((CHUNK,), _f32),
        pltpu.VMEM((RB, 16), _f32),
        pltpu.VMEM((RB,), _f32),
        pltpu.VMEM_SHARED((NP, 16), _f32),
        pltpu.VMEM_SHARED((NP,), _f32),
        pltpu.SemaphoreType.DMA,
        pltpu.SemaphoreType.DMA,
    ],
    compiler_params=_sc_params,
)


def _fin_body(pr_ref, pd_ref, g_ref, aa_ref, mx_ref, b2_ref, o_ref):
    m2 = _leaky(mx_ref[0, 0] + mx_ref[0, 1])
    aa = aa_ref[...]
    a2s = _leaky(aa[:, 0:1] + aa[:, 1:2])
    es = jnp.exp(a2s - m2)
    g = g_ref[...]
    num = pr_ref[0] + pr_ref[1] + es * g
    den = pd_ref[0] + pd_ref[1] + es + 1e-16
    o_ref[...] = num / den + b2_ref[...]


_fin = pl.pallas_call(
    _fin_body,
    grid=(GRID,),
    in_specs=[
        pl.BlockSpec((2, BT, 16), lambda i: (0, i, 0)),
        pl.BlockSpec((2, BT, 1), lambda i: (0, i, 0)),
        pl.BlockSpec((BT, 16), lambda i: (i, 0)),
        pl.BlockSpec((BT, 2), lambda i: (i, 0)),
        pl.BlockSpec((1, 16), lambda i: (0, 0)),
        pl.BlockSpec((1, 16), lambda i: (0, 0)),
    ],
    out_specs=pl.BlockSpec((BT, 16), lambda i: (i, 0)),
    out_shape=jax.ShapeDtypeStruct((N, 16), _f32),
)


def kernel(x, edge_index, W1, att_src1, att_dst1, b1, W2, att_src2, att_dst2,
           b2):
    src = edge_index[0].astype(_i32)
    dst = edge_index[1].astype(_i32)
    xf = x.reshape(N)
    W1r = W1.reshape(4, 32)
    s1 = (W1r * att_src1[0]).sum(-1)
    d1 = (W1r * att_dst1[0]).sum(-1)
    wvec = jnp.concatenate([s1, d1, jnp.zeros((8,), _f32)])
    z16 = jnp.zeros((RB, 16), _f32)
    z8 = jnp.zeros((RB, 8), _f32)
    z1 = jnp.zeros((RB,), _f32)

    part1 = _l1(xf, src, dst, wvec, z8)

    sd = jnp.stack([s1, d1])
    att2 = jnp.stack([att_src2[0, 0], att_dst2[0, 0]], axis=1)
    g, aa, mx = _mid(part1, x, sd, W1r, b1.reshape(4, 32), W2, att2)

    as2 = aa[:, 0] + 0.0
    ad2 = aa[:, 1] + 0.0
    part2r, part2d = _l2(g, as2, ad2, src, dst, mx.reshape(16), z16, z1)

    return _fin(part2r, part2d.reshape(2, NP, 1), g, aa, mx,
                b2.reshape(1, 16))
